# baseline (speedup 1.0000x reference)
import jax
import jax.numpy as jnp
from jax import lax
from jax.experimental import pallas as pl
from jax.experimental.pallas import tpu as pltpu
from jax.experimental.pallas import tpu_sc as plsc

N_COARSE = 128
N_FINE = 64
B = 100000
NW = 32
RBLK = 40
NBLK_TOTAL = B // RBLK
INV = 1.0 / N_COARSE


def _sc_body(near_hbm, far_hbm, w_hbm, uc_hbm, uf_hbm, uf2_hbm, out_hbm,
             near_v, far_v, w_v, uc_v, uf_v, uf2_v, cdf_v, out_v):
    c = lax.axis_index("c")
    s = lax.axis_index("s")
    wid = s * 2 + c
    nb = (NBLK_TOTAL - wid + NW - 1) // NW

    def block_body(k, carry0):
        base = (wid + k * NW) * RBLK
        pltpu.sync_copy(near_hbm.at[pl.ds(base, RBLK)], near_v.at[pl.ds(0, RBLK)])
        pltpu.sync_copy(far_hbm.at[pl.ds(base, RBLK)], far_v.at[pl.ds(0, RBLK)])
        pltpu.sync_copy(w_hbm.at[pl.ds(base, RBLK)], w_v)
        pltpu.sync_copy(uc_hbm.at[pl.ds(base, RBLK)], uc_v)
        pltpu.sync_copy(uf_hbm.at[pl.ds(base, RBLK)], uf_v)
        pltpu.sync_copy(uf2_hbm.at[pl.ds(base, RBLK)], uf2_v)

        @plsc.parallel_loop(0, RBLK, 1, unroll=4)
        def ray_body(r):
            near = near_v[pl.ds(r, 16)][0]
            far = far_v[pl.ds(r, 16)][0]
            scans = []
            for k8 in range(8):
                wk = w_v[r, pl.ds(16 * k8, 16)] + jnp.float32(1e-5)
                scans.append(plsc.cumsum(wk))
            total = jnp.float32(0.0)
            for k8 in range(8):
                ck = scans[k8] + total
                cdf_v[pl.ds(r * N_COARSE + 16 * k8, 16)] = ck
                total = total + scans[k8][15]
            for k8 in range(8):
                jf = lax.iota(jnp.int32, 16).astype(jnp.float32) + jnp.float32(16 * k8)
                zs = (jf + uc_v[r, pl.ds(16 * k8, 16)]) * jnp.float32(INV)
                out_v[r, pl.ds(16 * k8, 16)] = near * (1.0 - zs) + far * zs

            cbase = jnp.full((16,), r * N_COARSE, jnp.int32)
            for q in range(4):
                v = uf_v[r, pl.ds(16 * q, 16)] * total
                lo = jnp.zeros((16,), jnp.int32)
                hi = jnp.full((16,), N_COARSE, jnp.int32)
                for _ in range(7):
                    mid = (lo + hi) >> 1
                    cg = plsc.load_gather(cdf_v, [cbase + mid])
                    pred = cg <= v
                    lo = jnp.where(pred, mid + 1, lo)
                    hi = jnp.where(pred, hi, mid)
                zs = (lo.astype(jnp.float32) + uf2_v[r, pl.ds(16 * q, 16)]) * jnp.float32(INV)
                out_v[r, pl.ds(N_COARSE + 16 * q, 16)] = near * (1.0 - zs) + far * zs

        pltpu.sync_copy(out_v, out_hbm.at[pl.ds(base, RBLK)])
        return carry0

    lax.fori_loop(0, nb, block_body, 0)


def kernel(rays, weights, u_coarse, u_fine, u_fine2):
    near = rays[:, 6]
    far = rays[:, 7]
    mesh = plsc.VectorSubcoreMesh(core_axis_name="c", subcore_axis_name="s")
    f = pl.kernel(
        _sc_body,
        mesh=mesh,
        compiler_params=pltpu.CompilerParams(
            use_tc_tiling_on_sc=False, needs_layout_passes=False
        ),
        out_type=jax.ShapeDtypeStruct((B, N_COARSE + N_FINE), jnp.float32),
        scratch_types=[
            pltpu.VMEM((RBLK + 16,), jnp.float32),
            pltpu.VMEM((RBLK + 16,), jnp.float32),
            pltpu.VMEM((RBLK, N_COARSE), jnp.float32),
            pltpu.VMEM((RBLK, N_COARSE), jnp.float32),
            pltpu.VMEM((RBLK, N_FINE), jnp.float32),
            pltpu.VMEM((RBLK, N_FINE), jnp.float32),
            pltpu.VMEM((RBLK * N_COARSE,), jnp.float32),
            pltpu.VMEM((RBLK, N_COARSE + N_FINE), jnp.float32),
        ],
    )
    return f(near, far, weights, u_coarse, u_fine, u_fine2)


# speedup vs baseline: 1.0287x; 1.0287x over previous
import jax
import jax.numpy as jnp
from jax import lax
from jax.experimental import pallas as pl
from jax.experimental.pallas import tpu as pltpu
from jax.experimental.pallas import tpu_sc as plsc

N_COARSE = 128
N_FINE = 64
B = 100000
NW = 32
RBLK = 40
NBLK_TOTAL = B // RBLK
INV = 1.0 / N_COARSE


def _sc_body(near_hbm, far_hbm, w_hbm, uc_hbm, uf_hbm, uf2_hbm, out_hbm,
             near_v, far_v, w_v, uc_v, uf_v, uf2_v, cdf_v, out_v):
    c = lax.axis_index("c")
    s = lax.axis_index("s")
    wid = s * 2 + c
    nb = (NBLK_TOTAL - wid + NW - 1) // NW

    def block_body(k, carry0):
        base = (wid + k * NW) * RBLK
        pltpu.sync_copy(near_hbm.at[pl.ds(base, RBLK)], near_v.at[pl.ds(0, RBLK)])
        pltpu.sync_copy(far_hbm.at[pl.ds(base, RBLK)], far_v.at[pl.ds(0, RBLK)])
        pltpu.sync_copy(w_hbm.at[pl.ds(base, RBLK)], w_v)
        pltpu.sync_copy(uc_hbm.at[pl.ds(base, RBLK)], uc_v)
        pltpu.sync_copy(uf_hbm.at[pl.ds(base, RBLK)], uf_v)
        pltpu.sync_copy(uf2_hbm.at[pl.ds(base, RBLK)], uf2_v)

        @plsc.parallel_loop(0, RBLK, 1, unroll=2)
        def ray_body(r):
            near = near_v[pl.ds(r, 16)][0]
            far = far_v[pl.ds(r, 16)][0]
            scans = []
            for k8 in range(8):
                wk = w_v[r, pl.ds(16 * k8, 16)] + jnp.float32(1e-5)
                scans.append(plsc.cumsum(wk))
            total = jnp.float32(0.0)
            for k8 in range(8):
                ck = scans[k8] + total
                cdf_v[pl.ds(r * N_COARSE + 16 * k8, 16)] = ck
                total = total + scans[k8][15]
            for k8 in range(8):
                jf = lax.iota(jnp.int32, 16).astype(jnp.float32) + jnp.float32(16 * k8)
                zs = (jf + uc_v[r, pl.ds(16 * k8, 16)]) * jnp.float32(INV)
                out_v[r, pl.ds(16 * k8, 16)] = near * (1.0 - zs) + far * zs

            cbase = jnp.full((16,), r * N_COARSE, jnp.int32)
            for q in range(4):
                v = uf_v[r, pl.ds(16 * q, 16)] * total
                lo = jnp.zeros((16,), jnp.int32)
                hi = jnp.full((16,), N_COARSE, jnp.int32)
                for _ in range(7):
                    mid = (lo + hi) >> 1
                    cg = plsc.load_gather(cdf_v, [cbase + mid])
                    pred = cg <= v
                    lo = jnp.where(pred, mid + 1, lo)
                    hi = jnp.where(pred, hi, mid)
                zs = (lo.astype(jnp.float32) + uf2_v[r, pl.ds(16 * q, 16)]) * jnp.float32(INV)
                out_v[r, pl.ds(N_COARSE + 16 * q, 16)] = near * (1.0 - zs) + far * zs

        pltpu.sync_copy(out_v, out_hbm.at[pl.ds(base, RBLK)])
        return carry0

    lax.fori_loop(0, nb, block_body, 0)


def kernel(rays, weights, u_coarse, u_fine, u_fine2):
    near = rays[:, 6]
    far = rays[:, 7]
    mesh = plsc.VectorSubcoreMesh(core_axis_name="c", subcore_axis_name="s")
    f = pl.kernel(
        _sc_body,
        mesh=mesh,
        compiler_params=pltpu.CompilerParams(
            use_tc_tiling_on_sc=False, needs_layout_passes=False
        ),
        out_type=jax.ShapeDtypeStruct((B, N_COARSE + N_FINE), jnp.float32),
        scratch_types=[
            pltpu.VMEM((RBLK + 16,), jnp.float32),
            pltpu.VMEM((RBLK + 16,), jnp.float32),
            pltpu.VMEM((RBLK, N_COARSE), jnp.float32),
            pltpu.VMEM((RBLK, N_COARSE), jnp.float32),
            pltpu.VMEM((RBLK, N_FINE), jnp.float32),
            pltpu.VMEM((RBLK, N_FINE), jnp.float32),
            pltpu.VMEM((RBLK * N_COARSE,), jnp.float32),
            pltpu.VMEM((RBLK, N_COARSE + N_FINE), jnp.float32),
        ],
    )
    return f(near, far, weights, u_coarse, u_fine, u_fine2)


# speedup vs baseline: 1.4469x; 1.4066x over previous
import jax
import jax.numpy as jnp
from jax import lax
from jax.experimental import pallas as pl
from jax.experimental.pallas import tpu as pltpu
from jax.experimental.pallas import tpu_sc as plsc

N_COARSE = 128
N_FINE = 64
B = 100000
NW = 32
RBLK = 40
NBLK_TOTAL = B // RBLK
INV = 1.0 / N_COARSE


def _sc_body(near_hbm, far_hbm, w_hbm, uc_hbm, uf_hbm, uf2_hbm, out_hbm,
             near_v, far_v, w_v, uc_v, uf_v, uf2_v, cdf_v, out_v):
    c = lax.axis_index("c")
    s = lax.axis_index("s")
    wid = s * 2 + c
    nb = (NBLK_TOTAL - wid + NW - 1) // NW

    def block_body(k, carry0):
        base = (wid + k * NW) * RBLK
        pltpu.sync_copy(near_hbm.at[pl.ds(base, RBLK)], near_v.at[pl.ds(0, RBLK)])
        pltpu.sync_copy(far_hbm.at[pl.ds(base, RBLK)], far_v.at[pl.ds(0, RBLK)])
        pltpu.sync_copy(w_hbm.at[pl.ds(base, RBLK)], w_v)
        pltpu.sync_copy(uc_hbm.at[pl.ds(base, RBLK)], uc_v)
        pltpu.sync_copy(uf_hbm.at[pl.ds(base, RBLK)], uf_v)
        pltpu.sync_copy(uf2_hbm.at[pl.ds(base, RBLK)], uf2_v)

        @plsc.parallel_loop(0, RBLK, 1, unroll=2)
        def ray_body(r):
            near = near_v[pl.ds(r, 16)][0]
            far = far_v[pl.ds(r, 16)][0]
            scans = []
            for k8 in range(8):
                wk = w_v[r, pl.ds(16 * k8, 16)] + jnp.float32(1e-5)
                scans.append(plsc.cumsum(wk))
            total = jnp.float32(0.0)
            for k8 in range(8):
                ck = scans[k8] + total
                cdf_v[pl.ds(r * N_COARSE + 16 * k8, 16)] = ck
                total = total + scans[k8][15]
            for k8 in range(8):
                jf = lax.iota(jnp.int32, 16).astype(jnp.float32) + jnp.float32(16 * k8)
                zs = (jf + uc_v[r, pl.ds(16 * k8, 16)]) * jnp.float32(INV)
                out_v[r, pl.ds(16 * k8, 16)] = near * (1.0 - zs) + far * zs

            cbase = jnp.full((16,), r * N_COARSE, jnp.int32)
            for q in range(4):
                v = uf_v[r, pl.ds(16 * q, 16)] * total
                lo = jnp.zeros((16,), jnp.int32)
                hi = jnp.full((16,), N_COARSE, jnp.int32)
                for _ in range(7):
                    mid = (lo + hi) >> 1
                    cg = plsc.load_gather(cdf_v, [cbase + mid])
                    pred = cg <= v
                    lo = jnp.where(pred, mid + 1, lo)
                    hi = jnp.where(pred, hi, mid)
                zs = (lo.astype(jnp.float32) + uf2_v[r, pl.ds(16 * q, 16)]) * jnp.float32(INV)
                out_v[r, pl.ds(N_COARSE + 16 * q, 16)] = near * (1.0 - zs) + far * zs

        pltpu.sync_copy(out_v, out_hbm.at[pl.ds(base, RBLK)])
        return carry0

    lax.fori_loop(0, nb, block_body, 0)


def kernel(rays, weights, u_coarse, u_fine, u_fine2):
    near = rays[:, 6]
    far = rays[:, 7]
    mesh = plsc.VectorSubcoreMesh(core_axis_name="c", subcore_axis_name="s")
    f = pl.kernel(
        _sc_body,
        mesh=mesh,
        compiler_params=pltpu.CompilerParams(
            use_tc_tiling_on_sc=True, needs_layout_passes=False
        ),
        out_type=jax.ShapeDtypeStruct((B, N_COARSE + N_FINE), jnp.float32),
        scratch_types=[
            pltpu.VMEM((RBLK + 16,), jnp.float32),
            pltpu.VMEM((RBLK + 16,), jnp.float32),
            pltpu.VMEM((RBLK, N_COARSE), jnp.float32),
            pltpu.VMEM((RBLK, N_COARSE), jnp.float32),
            pltpu.VMEM((RBLK, N_FINE), jnp.float32),
            pltpu.VMEM((RBLK, N_FINE), jnp.float32),
            pltpu.VMEM((RBLK * N_COARSE,), jnp.float32),
            pltpu.VMEM((RBLK, N_COARSE + N_FINE), jnp.float32),
        ],
    )
    return f(near, far, weights, u_coarse, u_fine, u_fine2)
